# Initial kernel scaffold; baseline (speedup 1.0000x reference)
import jax
import jax.numpy as jnp
from jax.experimental import pallas as pl


def kernel(x, edge_index, We, be, W1a, b1a, W2a, b2a, W1b, b1b, W2b, b2b):
    raise NotImplementedError("write your pallas kernel here")


# speedup vs baseline: 11.3951x; 11.3951x over previous
import functools

import jax
import jax.numpy as jnp
from jax import lax
from jax.experimental import pallas as pl
from jax.experimental.pallas import tpu as pltpu
from jax.experimental.pallas import tpu_sc as plsc

NC = 2
NS = 16
NW = NC * NS
L = 16
CH = 80
TAU = 0.05

_MESH = plsc.VectorSubcoreMesh(core_axis_name="c", subcore_axis_name="s",
                               num_cores=NC, num_subcores=NS)
_SC_PARAMS = pltpu.CompilerParams(use_tc_tiling_on_sc=False,
                                  needs_layout_passes=False)


def _f32(*shape):
    return jax.ShapeDtypeStruct(shape, jnp.float32)


@functools.cache
def _sc_deg(n, ech):
    cpw = ech // NW

    @functools.partial(
        pl.kernel,
        out_type=_f32(NC, 1, n),
        mesh=_MESH,
        compiler_params=_SC_PARAMS,
        scratch_types=[
            pltpu.VMEM((cpw, CH), jnp.int32),
            pltpu.VMEM((CH,), jnp.float32),
            pltpu.VMEM_SHARED((n,), jnp.float32),
        ],
    )
    def deg_kernel(zeros_hbm, dst_hbm, degp_hbm, dstm, ones_v, acc):
        c = lax.axis_index("c")
        s = lax.axis_index("s")
        wid = s * NC + c
        for i in range(CH // L):
            ones_v[pl.ds(i * L, L)] = jnp.full((L,), 1.0, jnp.float32)

        @pl.when(s == 0)
        def _():
            pltpu.sync_copy(zeros_hbm, acc)

        plsc.subcore_barrier()
        pltpu.sync_copy(dst_hbm.at[wid], dstm)

        def chunk(j, carry):
            pltpu.sync_copy(ones_v, acc.at[dstm.at[j]], add=True)
            return carry

        lax.fori_loop(0, cpw, chunk, 0)
        plsc.subcore_barrier()

        @pl.when(s == 0)
        def _():
            pltpu.sync_copy(acc, degp_hbm.at[c, 0])

    return deg_kernel


@functools.cache
def _sc_spmm(n, ech, d, weighted):
    cpw = ech // NW
    rpn = n // NS

    scratch = [
        pltpu.VMEM((cpw, CH), jnp.int32),
        pltpu.VMEM((cpw, CH), jnp.int32),
        pltpu.VMEM((CH, d), jnp.float32),
        pltpu.VMEM_SHARED((n, d), jnp.float32),
        pltpu.SemaphoreType.DMA,
    ]
    if weighted:
        scratch.insert(2, pltpu.VMEM((cpw, CH), jnp.float32))

    @functools.partial(pl.kernel, out_type=_f32(NC * NS, rpn, d), mesh=_MESH,
                       compiler_params=_SC_PARAMS, scratch_types=scratch)
    def spmm_kernel(*refs):
        if weighted:
            (zeros_hbm, table_hbm, src_hbm, dst_hbm, w_hbm, part_hbm,
             srcm, dstm, wm, rows, acc, sem) = refs
        else:
            (zeros_hbm, table_hbm, src_hbm, dst_hbm, part_hbm,
             srcm, dstm, rows, acc, sem) = refs
        c = lax.axis_index("c")
        s = lax.axis_index("s")
        wid = s * NC + c

        @pl.when(s == 0)
        def _():
            pltpu.sync_copy(zeros_hbm, acc)

        plsc.subcore_barrier()
        pltpu.sync_copy(src_hbm.at[wid], srcm)
        pltpu.sync_copy(dst_hbm.at[wid], dstm)
        if weighted:
            pltpu.sync_copy(w_hbm.at[wid], wm)

        def chunk(j, carry):
            pltpu.async_copy(table_hbm.at[srcm.at[j]], rows, sem).wait()
            if weighted:
                def gscale(g, gcarry):
                    wvec = wm[j, pl.ds(g * L, L)]
                    for j2 in range(L):
                        e = g * L + j2
                        wv = wvec[j2]
                        for f in range(d // L):
                            rows[e, pl.ds(f * L, L)] = (
                                rows[e, pl.ds(f * L, L)] * wv)
                    return gcarry
                lax.fori_loop(0, CH // L, gscale, 0)
            pltpu.sync_copy(rows, acc.at[dstm.at[j]], add=True)
            return carry

        lax.fori_loop(0, cpw, chunk, 0)
        plsc.subcore_barrier()
        pltpu.sync_copy(acc.at[pl.ds(s * rpn, rpn)],
                        part_hbm.at[c * NS + s])

    return spmm_kernel


@functools.cache
def _sc_edges(n, ech, d):
    cpw = ech // NW

    @functools.partial(
        pl.kernel,
        out_type=(_f32(NW, cpw, CH), _f32(NC, 1, n)),
        mesh=_MESH,
        compiler_params=_SC_PARAMS,
        scratch_types=[
            pltpu.VMEM((cpw, CH), jnp.int32),
            pltpu.VMEM((cpw, CH), jnp.int32),
            pltpu.VMEM((CH, d), jnp.float32),
            pltpu.VMEM((CH, d), jnp.float32),
            pltpu.VMEM((cpw, CH), jnp.float32),
            pltpu.VMEM_SHARED((n,), jnp.float32),
            pltpu.SemaphoreType.DMA,
        ],
    )
    def edges_kernel(zeros_hbm, h_hbm, src_hbm, dst_hbm, ew_hbm, degp_hbm,
                     srcm, dstm, hs, hd, ewm, dacc, sem):
        c = lax.axis_index("c")
        s = lax.axis_index("s")
        wid = s * NC + c

        @pl.when(s == 0)
        def _():
            pltpu.sync_copy(zeros_hbm, dacc)

        plsc.subcore_barrier()
        pltpu.sync_copy(src_hbm.at[wid], srcm)
        pltpu.sync_copy(dst_hbm.at[wid], dstm)

        lane = jnp.arange(L, dtype=jnp.int32)

        def chunk(j, carry):
            pltpu.async_copy(h_hbm.at[srcm.at[j]], hs, sem).wait()
            pltpu.async_copy(h_hbm.at[dstm.at[j]], hd, sem).wait()

            def grp(g, gcarry):
                ridx = lane + g * L
                acc = jnp.zeros((L,), jnp.float32)
                for f in range(d):
                    cidx = jnp.full((L,), f, jnp.int32)
                    a = plsc.load_gather(hs, [ridx, cidx])
                    b = plsc.load_gather(hd, [ridx, cidx])
                    acc = acc + a * b
                acc = jnp.where(acc >= TAU, acc, 0.0)
                ewm[j, pl.ds(g * L, L)] = acc
                return gcarry

            lax.fori_loop(0, CH // L, grp, 0)
            pltpu.sync_copy(ewm.at[j], dacc.at[dstm.at[j]], add=True)
            return carry

        lax.fori_loop(0, cpw, chunk, 0)
        pltpu.sync_copy(ewm, ew_hbm.at[wid])
        plsc.subcore_barrier()

        @pl.when(s == 0)
        def _():
            pltpu.sync_copy(dacc, degp_hbm.at[c, 0])

    return edges_kernel


@functools.cache
def _tc_mm2(n, k, m1, m2, br):
    def body(x_ref, w1_ref, w2_ref, o1_ref, o2_ref):
        xb = x_ref[...]
        o1_ref[...] = jnp.dot(xb, w1_ref[...], preferred_element_type=jnp.float32)
        o2_ref[...] = jnp.dot(xb, w2_ref[...], preferred_element_type=jnp.float32)

    return pl.pallas_call(
        body,
        grid=(n // br,),
        in_specs=[
            pl.BlockSpec((br, k), lambda i: (i, 0)),
            pl.BlockSpec((k, m1), lambda i: (0, 0)),
            pl.BlockSpec((k, m2), lambda i: (0, 0)),
        ],
        out_specs=[
            pl.BlockSpec((br, m1), lambda i: (i, 0)),
            pl.BlockSpec((br, m2), lambda i: (i, 0)),
        ],
        out_shape=[_f32(n, m1), _f32(n, m2)],
    )


@functools.cache
def _tc_mm(n, k, m, br):
    def body(x_ref, w_ref, o_ref):
        o_ref[...] = jnp.dot(x_ref[...], w_ref[...],
                             preferred_element_type=jnp.float32)

    return pl.pallas_call(
        body,
        grid=(n // br,),
        in_specs=[
            pl.BlockSpec((br, k), lambda i: (i, 0)),
            pl.BlockSpec((k, m), lambda i: (0, 0)),
        ],
        out_specs=pl.BlockSpec((br, m), lambda i: (i, 0)),
        out_shape=_f32(n, m),
    )


@functools.cache
def _tc_h(n, d, br):
    def body(s_ref, t_ref, dis_ref, b_ref, o_ref):
        h = dis_ref[...] * (s_ref[...] + t_ref[...]) + b_ref[...]
        h = jnp.maximum(h, 0.0)
        hn = jnp.sqrt(jnp.sum(h * h, axis=1, keepdims=True))
        o_ref[...] = h / jnp.clip(hn, 1e-12, None)

    return pl.pallas_call(
        body,
        grid=(n // br,),
        in_specs=[
            pl.BlockSpec((br, d), lambda i: (i, 0)),
            pl.BlockSpec((br, d), lambda i: (i, 0)),
            pl.BlockSpec((br, 1), lambda i: (i, 0)),
            pl.BlockSpec((1, d), lambda i: (0, 0)),
        ],
        out_specs=pl.BlockSpec((br, d), lambda i: (i, 0)),
        out_shape=_f32(n, d),
    )


def kernel(x, edge_index, We, be, W1a, b1a, W2a, b2a, W1b, b1b, W2b, b2b):
    n, nfeat = x.shape
    e = edge_index.shape[1]
    nhid = We.shape[1]
    ncls = W2a.shape[1]
    ech = e // CH
    cpw = ech // NW
    br = 400

    src = edge_index[0].astype(jnp.int32).reshape(NW, cpw, CH)
    dst = edge_index[1].astype(jnp.int32).reshape(NW, cpw, CH)

    W1ab = jnp.concatenate([W1a, W1b], axis=1)
    b_ab = jnp.concatenate([b1a, b1b])
    d2 = 2 * nhid
    dz = 32
    W2blk = jnp.zeros((d2, dz), jnp.float32)
    W2blk = W2blk.at[:nhid, :ncls].set(W2a).at[nhid:, ncls:2 * ncls].set(W2b)

    zn = jnp.zeros((n,), jnp.float32)

    XW, XW2 = _tc_mm2(n, nfeat, nhid, d2, br)(x, We, W1ab)

    degp = _sc_deg(n, ech)(zn, dst)
    dis1 = lax.rsqrt(degp[0, 0] + degp[1, 0] + 1.0)
    T1 = XW * dis1[:, None]
    Sp = _sc_spmm(n, ech, nhid, False)(jnp.zeros((n, nhid), jnp.float32),
                                       T1, src, dst).reshape(NC, n, nhid)
    h = _tc_h(n, nhid, br)(Sp[0] + Sp[1], T1, dis1[:, None], be[None, :])

    ew2, d2p = _sc_edges(n, ech, nhid)(zn, h, src, dst)
    deg2 = d2p[0, 0] + d2p[1, 0]
    dis2 = jnp.where(deg2 > 0, lax.rsqrt(jnp.where(deg2 > 0, deg2, 1.0)), 0.0)

    T2 = XW2 * dis2[:, None]
    Up = _sc_spmm(n, ech, d2, True)(jnp.zeros((n, d2), jnp.float32),
                                    T2, src, dst, ew2).reshape(NC, n, d2)
    H1 = jax.nn.relu(dis2[:, None] * (Up[0] + Up[1]) + b_ab)
    Z = _tc_mm(n, d2, dz, br)(H1, W2blk)
    T3 = Z * dis2[:, None]
    Vp = _sc_spmm(n, ech, dz, True)(jnp.zeros((n, dz), jnp.float32),
                                    T3, src, dst, ew2).reshape(NC, n, dz)
    outk = dis2[:, None] * (Vp[0] + Vp[1])
    out1 = outk[:, :ncls] + b2a
    out2 = outk[:, ncls:2 * ncls] + b2b
    return (out1, out2)


# speedup vs baseline: 18.6297x; 1.6349x over previous
import functools

import jax
import jax.numpy as jnp
from jax import lax
from jax.experimental import pallas as pl
from jax.experimental.pallas import tpu as pltpu
from jax.experimental.pallas import tpu_sc as plsc

NC = 2
NS = 16
NW = NC * NS
L = 16
CH = 80
TAU = 0.05

_MESH = plsc.VectorSubcoreMesh(core_axis_name="c", subcore_axis_name="s",
                               num_cores=NC, num_subcores=NS)
_SC_PARAMS = pltpu.CompilerParams(use_tc_tiling_on_sc=False,
                                  needs_layout_passes=False)


def _f32(*shape):
    return jax.ShapeDtypeStruct(shape, jnp.float32)


@functools.cache
def _sc_deg(n, ech):
    cpw = ech // NW

    @functools.partial(
        pl.kernel,
        out_type=_f32(NC, 1, n),
        mesh=_MESH,
        compiler_params=_SC_PARAMS,
        scratch_types=[
            pltpu.VMEM((cpw, CH), jnp.int32),
            pltpu.VMEM((CH,), jnp.float32),
            pltpu.VMEM_SHARED((n,), jnp.float32),
        ],
    )
    def deg_kernel(zeros_hbm, dst_hbm, degp_hbm, dstm, ones_v, acc):
        c = lax.axis_index("c")
        s = lax.axis_index("s")
        wid = s * NC + c
        for i in range(CH // L):
            ones_v[pl.ds(i * L, L)] = jnp.full((L,), 1.0, jnp.float32)

        @pl.when(s == 0)
        def _():
            pltpu.sync_copy(zeros_hbm, acc)

        plsc.subcore_barrier()
        pltpu.sync_copy(dst_hbm.at[wid], dstm)

        def chunk(j, carry):
            pltpu.sync_copy(ones_v, acc.at[dstm.at[j]], add=True)
            return carry

        lax.fori_loop(0, cpw, chunk, 0)
        plsc.subcore_barrier()

        @pl.when(s == 0)
        def _():
            pltpu.sync_copy(acc, degp_hbm.at[c, 0])

    return deg_kernel


@functools.cache
def _sc_spmm(n, ech, d, weighted):
    cpw = ech // NW
    rpn = n // NS
    dp = d // 2

    assert cpw % 2 == 1
    npair = (cpw - 1) // 2
    scratch = [
        pltpu.VMEM((cpw, CH), jnp.int32),
        pltpu.VMEM((cpw, CH), jnp.int32),
        pltpu.VMEM((CH, dp), jnp.int32),
        pltpu.VMEM((CH, dp), jnp.int32),
        pltpu.VMEM((CH, d), jnp.float32),
        pltpu.VMEM((CH, d), jnp.float32),
        pltpu.VMEM_SHARED((n, d), jnp.float32),
        pltpu.SemaphoreType.DMA,
        pltpu.SemaphoreType.DMA,
        pltpu.SemaphoreType.DMA,
        pltpu.SemaphoreType.DMA,
    ]
    if weighted:
        scratch.insert(2, pltpu.VMEM((cpw, CH), jnp.float32))

    @functools.partial(pl.kernel, out_type=_f32(NC * NS, rpn, d), mesh=_MESH,
                       compiler_params=_SC_PARAMS, scratch_types=scratch)
    def spmm_kernel(*refs):
        if weighted:
            (zeros_hbm, table_hbm, src_hbm, dst_hbm, w_hbm, part_hbm,
             srcm, dstm, wm, grows0, grows1, rows0, rows1, acc,
             gsem0, gsem1, ssem0, ssem1) = refs
        else:
            (zeros_hbm, table_hbm, src_hbm, dst_hbm, part_hbm,
             srcm, dstm, grows0, grows1, rows0, rows1, acc,
             gsem0, gsem1, ssem0, ssem1) = refs
        c = lax.axis_index("c")
        s = lax.axis_index("s")
        wid = s * NC + c
        gbufs = (grows0, grows1)
        fbufs = (rows0, rows1)
        gsems = (gsem0, gsem1)
        ssems = (ssem0, ssem1)

        @pl.when(s == 0)
        def _():
            pltpu.sync_copy(zeros_hbm, acc)

        plsc.subcore_barrier()
        pltpu.sync_copy(src_hbm.at[wid], srcm)
        pltpu.sync_copy(dst_hbm.at[wid], dstm)
        if weighted:
            pltpu.sync_copy(w_hbm.at[wid], wm)

        def start(j, b):
            pltpu.async_copy(table_hbm.at[srcm.at[j]], gbufs[b], gsems[b])

        def finish(j, b):
            pltpu.make_async_copy(table_hbm.at[srcm.at[j]], gbufs[b],
                                  gsems[b]).wait()

        def process(j, b):
            grows = gbufs[b]
            rows = fbufs[b]

            def gbody(g, gcarry):
                if weighted:
                    wvec = wm[j, pl.ds(g * L, L)]
                for j2 in range(L):
                    e = g * L + j2
                    if weighted:
                        wv = wvec[j2]
                    for f in range(dp // L):
                        a, b2 = plsc.unpack(
                            plsc.bitcast(grows[e, pl.ds(f * L, L)],
                                         jnp.bfloat16),
                            format=plsc.PackFormat.INTERLEAVED)
                        if weighted:
                            a = a * wv
                            b2 = b2 * wv
                        rows[e, pl.ds(2 * f * L, L)] = a
                        rows[e, pl.ds((2 * f + 1) * L, L)] = b2
                return gcarry

            lax.fori_loop(0, CH // L, gbody, 0)

        def start_scatter(j, b):
            pltpu.async_copy(fbufs[b], acc.at[dstm.at[j]], ssems[b],
                             add=True)

        def wait_scatter(j, b):
            pltpu.make_async_copy(fbufs[b], acc.at[dstm.at[j]],
                                  ssems[b]).wait()

        start(0, 0)
        start(1, 1)

        def pair(k, carry):
            j0 = 2 * k

            finish(j0, 0)

            @pl.when(k > 0)
            def _():
                wait_scatter(j0, 0)

            process(j0, 0)
            start_scatter(j0, 0)
            start(j0 + 2, 0)
            finish(j0 + 1, 1)

            @pl.when(k > 0)
            def _():
                wait_scatter(j0 + 1, 1)

            process(j0 + 1, 1)
            start_scatter(j0 + 1, 1)

            @pl.when(k < npair - 1)
            def _():
                start(j0 + 3, 1)

            return carry

        lax.fori_loop(0, npair, pair, 0)
        finish(cpw - 1, 0)
        wait_scatter(cpw - 1, 0)
        process(cpw - 1, 0)
        pltpu.sync_copy(fbufs[0], acc.at[dstm.at[cpw - 1]], add=True)
        wait_scatter(cpw - 2, 1)
        plsc.subcore_barrier()
        pltpu.sync_copy(acc.at[pl.ds(s * rpn, rpn)],
                        part_hbm.at[c * NS + s])

    return spmm_kernel


@functools.cache
def _sc_edges(n, ech, d):
    cpw = ech // NW
    dp = d // 2

    @functools.partial(
        pl.kernel,
        out_type=(_f32(NW, cpw, CH), _f32(NC, 1, n)),
        mesh=_MESH,
        compiler_params=_SC_PARAMS,
        scratch_types=[
            pltpu.VMEM((cpw, CH), jnp.int32),
            pltpu.VMEM((cpw, CH), jnp.int32),
            pltpu.VMEM((CH, dp), jnp.int32),
            pltpu.VMEM((CH, dp), jnp.int32),
            pltpu.VMEM((CH, dp), jnp.int32),
            pltpu.VMEM((CH, dp), jnp.int32),
            pltpu.VMEM((cpw, CH), jnp.float32),
            pltpu.VMEM((L * 17,), jnp.float32),
            pltpu.VMEM_SHARED((n,), jnp.float32),
            pltpu.SemaphoreType.DMA,
            pltpu.SemaphoreType.DMA,
        ],
    )
    def edges_kernel(zeros_hbm, h_hbm, src_hbm, dst_hbm, ew_hbm, degp_hbm,
                     srcm, dstm, hs0, hd0, hs1, hd1, ewm, tbuf, dacc,
                     sem0, sem1):
        c = lax.axis_index("c")
        s = lax.axis_index("s")
        wid = s * NC + c
        sbufs = (hs0, hs1)
        dbufs = (hd0, hd1)
        sems = (sem0, sem1)

        @pl.when(s == 0)
        def _():
            pltpu.sync_copy(zeros_hbm, dacc)

        plsc.subcore_barrier()
        pltpu.sync_copy(src_hbm.at[wid], srcm)
        pltpu.sync_copy(dst_hbm.at[wid], dstm)

        lane = jnp.arange(L, dtype=jnp.int32)

        def start(j, b):
            pltpu.async_copy(h_hbm.at[srcm.at[j]], sbufs[b], sems[b])
            pltpu.async_copy(h_hbm.at[dstm.at[j]], dbufs[b], sems[b])

        def finish(j, b):
            pltpu.make_async_copy(h_hbm.at[srcm.at[j]], sbufs[b],
                                  sems[b]).wait()
            pltpu.make_async_copy(h_hbm.at[dstm.at[j]], dbufs[b],
                                  sems[b]).wait()

        lane17 = lane * 17

        def process(j, b):
            hs = sbufs[b]
            hd = dbufs[b]

            def grp(g, gcarry):
                for j2 in range(L):
                    e = g * L + j2
                    p = None
                    for f in range(dp // L):
                        sa, sb = plsc.unpack(
                            plsc.bitcast(hs[e, pl.ds(f * L, L)],
                                         jnp.bfloat16),
                            format=plsc.PackFormat.INTERLEAVED)
                        da, db = plsc.unpack(
                            plsc.bitcast(hd[e, pl.ds(f * L, L)],
                                         jnp.bfloat16),
                            format=plsc.PackFormat.INTERLEAVED)
                        q = sa * da + sb * db
                        p = q if p is None else p + q
                    plsc.store_scatter(tbuf, [lane17 + j2], p)
                acc = plsc.load_gather(tbuf, [lane])
                for k in range(1, L):
                    acc = acc + plsc.load_gather(tbuf, [lane + k * 17])
                acc = jnp.where(acc >= TAU, acc, 0.0)
                ewm[j, pl.ds(g * L, L)] = acc
                return gcarry

            lax.fori_loop(0, CH // L, grp, 0)
            pltpu.sync_copy(ewm.at[j], dacc.at[dstm.at[j]], add=True)

        start(0, 0)

        def pair(k, carry):
            j0 = 2 * k
            start(j0 + 1, 1)
            finish(j0, 0)
            process(j0, 0)
            start(j0 + 2, 0)
            finish(j0 + 1, 1)
            process(j0 + 1, 1)
            return carry

        lax.fori_loop(0, (cpw - 1) // 2, pair, 0)
        finish(cpw - 1, 0)
        process(cpw - 1, 0)
        pltpu.sync_copy(ewm, ew_hbm.at[wid])
        plsc.subcore_barrier()

        @pl.when(s == 0)
        def _():
            pltpu.sync_copy(dacc, degp_hbm.at[c, 0])

    return edges_kernel


@functools.cache
def _tc_mm2(n, k, m1, m2, br):
    def body(x_ref, w1_ref, w2_ref, o1_ref, o2_ref):
        xb = x_ref[...]
        o1_ref[...] = jnp.dot(xb, w1_ref[...], preferred_element_type=jnp.float32)
        o2_ref[...] = jnp.dot(xb, w2_ref[...], preferred_element_type=jnp.float32)

    return pl.pallas_call(
        body,
        grid=(n // br,),
        in_specs=[
            pl.BlockSpec((br, k), lambda i: (i, 0)),
            pl.BlockSpec((k, m1), lambda i: (0, 0)),
            pl.BlockSpec((k, m2), lambda i: (0, 0)),
        ],
        out_specs=[
            pl.BlockSpec((br, m1), lambda i: (i, 0)),
            pl.BlockSpec((br, m2), lambda i: (i, 0)),
        ],
        out_shape=[_f32(n, m1), _f32(n, m2)],
    )


@functools.cache
def _tc_mm(n, k, m, br):
    def body(x_ref, w_ref, o_ref):
        o_ref[...] = jnp.dot(x_ref[...], w_ref[...],
                             preferred_element_type=jnp.float32)

    return pl.pallas_call(
        body,
        grid=(n // br,),
        in_specs=[
            pl.BlockSpec((br, k), lambda i: (i, 0)),
            pl.BlockSpec((k, m), lambda i: (0, 0)),
        ],
        out_specs=pl.BlockSpec((br, m), lambda i: (i, 0)),
        out_shape=_f32(n, m),
    )


@functools.cache
def _tc_h(n, d, br):
    def body(s_ref, t_ref, dis_ref, b_ref, o_ref):
        h = dis_ref[...] * (s_ref[...] + t_ref[...]) + b_ref[...]
        h = jnp.maximum(h, 0.0)
        hn = jnp.sqrt(jnp.sum(h * h, axis=1, keepdims=True))
        o_ref[...] = h / jnp.clip(hn, 1e-12, None)

    return pl.pallas_call(
        body,
        grid=(n // br,),
        in_specs=[
            pl.BlockSpec((br, d), lambda i: (i, 0)),
            pl.BlockSpec((br, d), lambda i: (i, 0)),
            pl.BlockSpec((br, 1), lambda i: (i, 0)),
            pl.BlockSpec((1, d), lambda i: (0, 0)),
        ],
        out_specs=pl.BlockSpec((br, d), lambda i: (i, 0)),
        out_shape=_f32(n, d),
    )


def kernel(x, edge_index, We, be, W1a, b1a, W2a, b2a, W1b, b1b, W2b, b2b):
    n, nfeat = x.shape
    e = edge_index.shape[1]
    nhid = We.shape[1]
    ncls = W2a.shape[1]
    ech = e // CH
    cpw = ech // NW
    br = 400

    src = edge_index[0].astype(jnp.int32).reshape(NW, cpw, CH)
    dst = edge_index[1].astype(jnp.int32).reshape(NW, cpw, CH)

    W1ab = jnp.concatenate([W1a, W1b], axis=1)
    b_ab = jnp.concatenate([b1a, b1b])
    d2 = 2 * nhid
    dz = 32
    W2blk = jnp.zeros((d2, dz), jnp.float32)
    W2blk = W2blk.at[:nhid, :ncls].set(W2a).at[nhid:, ncls:2 * ncls].set(W2b)

    zn = jnp.zeros((n,), jnp.float32)

    def _pack(t):
        m = t.shape[1]
        return jax.lax.bitcast_convert_type(
            t.astype(jnp.bfloat16).reshape(n, m // 2, 2), jnp.int32)

    def _unperm(u):
        m = u.shape[1]
        return u.reshape(n, m // 32, 2, 16).transpose(0, 1, 3, 2).reshape(
            n, m)

    XW, XW2 = _tc_mm2(n, nfeat, nhid, d2, br)(x, We, W1ab)

    degp = _sc_deg(n, ech)(zn, dst)
    dis1 = lax.rsqrt(degp[0, 0] + degp[1, 0] + 1.0)
    T1 = XW * dis1[:, None]
    Sp = _sc_spmm(n, ech, nhid, False)(jnp.zeros((n, nhid), jnp.float32),
                                       _pack(T1), src, dst
                                       ).reshape(NC, n, nhid)
    h = _tc_h(n, nhid, br)(_unperm(Sp[0] + Sp[1]), T1, dis1[:, None],
                           be[None, :])

    hp = jax.lax.bitcast_convert_type(
        h.astype(jnp.bfloat16).reshape(n, nhid // 2, 2), jnp.int32)
    ew2, d2p = _sc_edges(n, ech, nhid)(zn, hp, src, dst)
    deg2 = d2p[0, 0] + d2p[1, 0]
    dis2 = jnp.where(deg2 > 0, lax.rsqrt(jnp.where(deg2 > 0, deg2, 1.0)), 0.0)

    T2 = XW2 * dis2[:, None]
    spmm_w64 = _sc_spmm(n, ech, nhid, True)
    zn64 = jnp.zeros((n, nhid), jnp.float32)
    Upa = spmm_w64(zn64, _pack(T2[:, :nhid]), src, dst, ew2
                   ).reshape(NC, n, nhid)
    Upb = spmm_w64(zn64, _pack(T2[:, nhid:]), src, dst, ew2
                   ).reshape(NC, n, nhid)
    U = jnp.concatenate([_unperm(Upa[0] + Upa[1]),
                         _unperm(Upb[0] + Upb[1])], axis=1)
    H1 = jax.nn.relu(dis2[:, None] * U + b_ab)
    Z = _tc_mm(n, d2, dz, br)(H1, W2blk)
    T3 = Z * dis2[:, None]
    Vp = _sc_spmm(n, ech, dz, True)(jnp.zeros((n, dz), jnp.float32),
                                    _pack(T3), src, dst, ew2
                                    ).reshape(NC, n, dz)
    outk = dis2[:, None] * _unperm(Vp[0] + Vp[1])
    out1 = outk[:, :ncls] + b2a
    out2 = outk[:, ncls:2 * ncls] + b2b
    return (out1, out2)


# speedup vs baseline: 28.6778x; 1.5394x over previous
import functools

import jax
import jax.numpy as jnp
from jax import lax
from jax.experimental import pallas as pl
from jax.experimental.pallas import tpu as pltpu
from jax.experimental.pallas import tpu_sc as plsc

NC = 2
NS = 16
NW = NC * NS
L = 16
CH = 80
TAU = 0.05

_MESH = plsc.VectorSubcoreMesh(core_axis_name="c", subcore_axis_name="s",
                               num_cores=NC, num_subcores=NS)
_SC_PARAMS = pltpu.CompilerParams(use_tc_tiling_on_sc=False,
                                  needs_layout_passes=False)


def _f32(*shape):
    return jax.ShapeDtypeStruct(shape, jnp.float32)


@functools.cache
def _sc_deg(n, ech):
    cpw = ech // NW

    @functools.partial(
        pl.kernel,
        out_type=_f32(NC, 1, n),
        mesh=_MESH,
        compiler_params=_SC_PARAMS,
        scratch_types=[
            pltpu.VMEM((cpw, CH), jnp.int32),
            pltpu.VMEM((CH,), jnp.float32),
            pltpu.VMEM_SHARED((n,), jnp.float32),
        ],
    )
    def deg_kernel(zeros_hbm, dst_hbm, degp_hbm, dstm, ones_v, acc):
        c = lax.axis_index("c")
        s = lax.axis_index("s")
        wid = s * NC + c
        for i in range(CH // L):
            ones_v[pl.ds(i * L, L)] = jnp.full((L,), 1.0, jnp.float32)

        @pl.when(s == 0)
        def _():
            pltpu.sync_copy(zeros_hbm, acc)

        plsc.subcore_barrier()
        pltpu.sync_copy(dst_hbm.at[wid], dstm)

        def chunk(j, carry):
            pltpu.sync_copy(ones_v, acc.at[dstm.at[j]], add=True)
            return carry

        lax.fori_loop(0, cpw, chunk, 0)
        plsc.subcore_barrier()

        @pl.when(s == 0)
        def _():
            pltpu.sync_copy(acc, degp_hbm.at[c, 0])

    return deg_kernel


@functools.cache
def _sc_spmm(n, ech, d, weighted):
    cpw = ech // NW
    rpn = n // NS

    assert cpw % 2 == 1
    npair = (cpw - 1) // 2
    scratch = [
        pltpu.VMEM((cpw, CH), jnp.int32),
        pltpu.VMEM((cpw, CH), jnp.int32),
        pltpu.VMEM((CH, d), jnp.float32),
        pltpu.VMEM((CH, d), jnp.float32),
        pltpu.VMEM_SHARED((n, d), jnp.float32),
        pltpu.SemaphoreType.DMA,
        pltpu.SemaphoreType.DMA,
        pltpu.SemaphoreType.DMA,
        pltpu.SemaphoreType.DMA,
    ]
    if weighted:
        scratch.insert(2, pltpu.VMEM((cpw, CH), jnp.float32))

    @functools.partial(pl.kernel, out_type=_f32(NC, n, d), mesh=_MESH,
                       compiler_params=_SC_PARAMS, scratch_types=scratch)
    def spmm_kernel(*refs):
        if weighted:
            (zeros_hbm, table_hbm, src_hbm, dst_hbm, w_hbm, part_hbm,
             srcm, dstm, wm, rows0, rows1, acc,
             gsem0, gsem1, ssem0, ssem1) = refs
        else:
            (zeros_hbm, table_hbm, src_hbm, dst_hbm, part_hbm,
             srcm, dstm, rows0, rows1, acc,
             gsem0, gsem1, ssem0, ssem1) = refs
        c = lax.axis_index("c")
        s = lax.axis_index("s")
        wid = s * NC + c
        bufs = (rows0, rows1)
        gsems = (gsem0, gsem1)
        ssems = (ssem0, ssem1)

        @pl.when(s == 0)
        def _():
            pltpu.sync_copy(zeros_hbm, acc)

        plsc.subcore_barrier()
        pltpu.sync_copy(src_hbm.at[wid], srcm)
        pltpu.sync_copy(dst_hbm.at[wid], dstm)
        if weighted:
            pltpu.sync_copy(w_hbm.at[wid], wm)

        def start(j, b):
            pltpu.async_copy(table_hbm.at[srcm.at[j]], bufs[b], gsems[b])

        def finish(j, b):
            pltpu.make_async_copy(table_hbm.at[srcm.at[j]], bufs[b],
                                  gsems[b]).wait()

        def scale(j, b):
            rows = bufs[b]
            if weighted:
                def gscale(g, gcarry):
                    wvec = wm[j, pl.ds(g * L, L)]
                    for j2 in range(L):
                        e = g * L + j2
                        wv = wvec[j2]
                        for f in range(d // L):
                            rows[e, pl.ds(f * L, L)] = (
                                rows[e, pl.ds(f * L, L)] * wv)
                    return gcarry
                lax.fori_loop(0, CH // L, gscale, 0)

        def start_scatter(j, b):
            pltpu.async_copy(bufs[b], acc.at[dstm.at[j]], ssems[b], add=True)

        def wait_scatter(j, b):
            pltpu.make_async_copy(bufs[b], acc.at[dstm.at[j]],
                                  ssems[b]).wait()

        start(0, 0)
        start(1, 1)

        def pair(k, carry):
            j0 = 2 * k
            finish(j0, 0)
            scale(j0, 0)
            start_scatter(j0, 0)
            finish(j0 + 1, 1)
            scale(j0 + 1, 1)
            start_scatter(j0 + 1, 1)
            wait_scatter(j0, 0)
            start(j0 + 2, 0)
            wait_scatter(j0 + 1, 1)

            @pl.when(k < npair - 1)
            def _():
                start(j0 + 3, 1)

            return carry

        lax.fori_loop(0, npair, pair, 0)
        finish(cpw - 1, 0)
        scale(cpw - 1, 0)
        pltpu.sync_copy(bufs[0], acc.at[dstm.at[cpw - 1]], add=True)
        plsc.subcore_barrier()
        low = n - (NS - 1) * (rpn - 1)

        @pl.when(s < NS - 1)
        def _():
            pltpu.sync_copy(acc.at[pl.ds(s * (rpn - 1), rpn - 1)],
                            part_hbm.at[c, pl.ds(s * (rpn - 1), rpn - 1)])

        @pl.when(s == NS - 1)
        def _():
            pltpu.sync_copy(acc.at[pl.ds((NS - 1) * (rpn - 1), low)],
                            part_hbm.at[c, pl.ds((NS - 1) * (rpn - 1), low)])

    return spmm_kernel


@functools.cache
def _sc_edges(n, ech, d):
    cpw = ech // NW
    dp = d // 2

    @functools.partial(
        pl.kernel,
        out_type=(_f32(NW, cpw, CH), _f32(NC, 1, n)),
        mesh=_MESH,
        compiler_params=_SC_PARAMS,
        scratch_types=[
            pltpu.VMEM((cpw, CH), jnp.int32),
            pltpu.VMEM((cpw, CH), jnp.int32),
            pltpu.VMEM((CH, dp), jnp.int32),
            pltpu.VMEM((CH, dp), jnp.int32),
            pltpu.VMEM((CH, dp), jnp.int32),
            pltpu.VMEM((CH, dp), jnp.int32),
            pltpu.VMEM((cpw, CH), jnp.float32),
            pltpu.VMEM((L * 17,), jnp.float32),
            pltpu.VMEM_SHARED((n,), jnp.float32),
            pltpu.SemaphoreType.DMA,
            pltpu.SemaphoreType.DMA,
        ],
    )
    def edges_kernel(zeros_hbm, h_hbm, src_hbm, dst_hbm, ew_hbm, degp_hbm,
                     srcm, dstm, hs0, hd0, hs1, hd1, ewm, tbuf, dacc,
                     sem0, sem1):
        c = lax.axis_index("c")
        s = lax.axis_index("s")
        wid = s * NC + c
        sbufs = (hs0, hs1)
        dbufs = (hd0, hd1)
        sems = (sem0, sem1)

        @pl.when(s == 0)
        def _():
            pltpu.sync_copy(zeros_hbm, dacc)

        plsc.subcore_barrier()
        pltpu.sync_copy(src_hbm.at[wid], srcm)
        pltpu.sync_copy(dst_hbm.at[wid], dstm)

        lane = jnp.arange(L, dtype=jnp.int32)

        def start(j, b):
            pltpu.async_copy(h_hbm.at[srcm.at[j]], sbufs[b], sems[b])
            pltpu.async_copy(h_hbm.at[dstm.at[j]], dbufs[b], sems[b])

        def finish(j, b):
            pltpu.make_async_copy(h_hbm.at[srcm.at[j]], sbufs[b],
                                  sems[b]).wait()
            pltpu.make_async_copy(h_hbm.at[dstm.at[j]], dbufs[b],
                                  sems[b]).wait()

        lane17 = lane * 17

        def process(j, b):
            hs = sbufs[b]
            hd = dbufs[b]

            def grp(g, gcarry):
                for j2 in range(L):
                    e = g * L + j2
                    p = None
                    for f in range(dp // L):
                        sa, sb = plsc.unpack(
                            plsc.bitcast(hs[e, pl.ds(f * L, L)],
                                         jnp.bfloat16),
                            format=plsc.PackFormat.INTERLEAVED)
                        da, db = plsc.unpack(
                            plsc.bitcast(hd[e, pl.ds(f * L, L)],
                                         jnp.bfloat16),
                            format=plsc.PackFormat.INTERLEAVED)
                        q = sa * da + sb * db
                        p = q if p is None else p + q
                    plsc.store_scatter(tbuf, [lane17 + j2], p)
                acc = plsc.load_gather(tbuf, [lane])
                for k in range(1, L):
                    acc = acc + plsc.load_gather(tbuf, [lane + k * 17])
                acc = jnp.where(acc >= TAU, acc, 0.0)
                ewm[j, pl.ds(g * L, L)] = acc
                return gcarry

            lax.fori_loop(0, CH // L, grp, 0)
            pltpu.sync_copy(ewm.at[j], dacc.at[dstm.at[j]], add=True)

        start(0, 0)

        def pair(k, carry):
            j0 = 2 * k
            start(j0 + 1, 1)
            finish(j0, 0)
            process(j0, 0)
            start(j0 + 2, 0)
            finish(j0 + 1, 1)
            process(j0 + 1, 1)
            return carry

        lax.fori_loop(0, (cpw - 1) // 2, pair, 0)
        finish(cpw - 1, 0)
        process(cpw - 1, 0)
        pltpu.sync_copy(ewm, ew_hbm.at[wid])
        plsc.subcore_barrier()

        @pl.when(s == 0)
        def _():
            pltpu.sync_copy(dacc, degp_hbm.at[c, 0])

    return edges_kernel


@functools.cache
def _tc_mm2(n, k, m1, m2, br):
    def body(x_ref, w1_ref, w2_ref, o1_ref, o2_ref):
        xb = x_ref[...]
        o1_ref[...] = jnp.dot(xb, w1_ref[...], preferred_element_type=jnp.float32)
        o2_ref[...] = jnp.dot(xb, w2_ref[...], preferred_element_type=jnp.float32)

    return pl.pallas_call(
        body,
        grid=(n // br,),
        in_specs=[
            pl.BlockSpec((br, k), lambda i: (i, 0)),
            pl.BlockSpec((k, m1), lambda i: (0, 0)),
            pl.BlockSpec((k, m2), lambda i: (0, 0)),
        ],
        out_specs=[
            pl.BlockSpec((br, m1), lambda i: (i, 0)),
            pl.BlockSpec((br, m2), lambda i: (i, 0)),
        ],
        out_shape=[_f32(n, m1), _f32(n, m2)],
    )


@functools.cache
def _tc_mm(n, k, m, br):
    def body(x_ref, w_ref, o_ref):
        o_ref[...] = jnp.dot(x_ref[...], w_ref[...],
                             preferred_element_type=jnp.float32)

    return pl.pallas_call(
        body,
        grid=(n // br,),
        in_specs=[
            pl.BlockSpec((br, k), lambda i: (i, 0)),
            pl.BlockSpec((k, m), lambda i: (0, 0)),
        ],
        out_specs=pl.BlockSpec((br, m), lambda i: (i, 0)),
        out_shape=_f32(n, m),
    )


@functools.cache
def _tc_h(n, d, br):
    def body(s_ref, t_ref, dis_ref, b_ref, o_ref):
        h = dis_ref[...] * (s_ref[...] + t_ref[...]) + b_ref[...]
        h = jnp.maximum(h, 0.0)
        hn = jnp.sqrt(jnp.sum(h * h, axis=1, keepdims=True))
        o_ref[...] = h / jnp.clip(hn, 1e-12, None)

    return pl.pallas_call(
        body,
        grid=(n // br,),
        in_specs=[
            pl.BlockSpec((br, d), lambda i: (i, 0)),
            pl.BlockSpec((br, d), lambda i: (i, 0)),
            pl.BlockSpec((br, 1), lambda i: (i, 0)),
            pl.BlockSpec((1, d), lambda i: (0, 0)),
        ],
        out_specs=pl.BlockSpec((br, d), lambda i: (i, 0)),
        out_shape=_f32(n, d),
    )


def kernel(x, edge_index, We, be, W1a, b1a, W2a, b2a, W1b, b1b, W2b, b2b):
    n, nfeat = x.shape
    e = edge_index.shape[1]
    nhid = We.shape[1]
    ncls = W2a.shape[1]
    ech = e // CH
    cpw = ech // NW
    br = 10000

    src = edge_index[0].astype(jnp.int32).reshape(NW, cpw, CH)
    dst = edge_index[1].astype(jnp.int32).reshape(NW, cpw, CH)

    W1ab = jnp.concatenate([W1a, W1b], axis=1)
    b_ab = jnp.concatenate([b1a, b1b])
    d2 = 2 * nhid
    dz = 32
    W2blk = jnp.zeros((d2, dz), jnp.float32)
    W2blk = W2blk.at[:nhid, :ncls].set(W2a).at[nhid:, ncls:2 * ncls].set(W2b)

    zn = jnp.zeros((n,), jnp.float32)

    XW, XW2 = _tc_mm2(n, nfeat, nhid, d2, br)(x, We, W1ab)

    degp = _sc_deg(n, ech)(zn, dst)
    dis1 = lax.rsqrt(degp[0, 0] + degp[1, 0] + 1.0)
    T1 = XW * dis1[:, None]
    Sp = _sc_spmm(n, ech, nhid, False)(jnp.zeros((n, nhid), jnp.float32),
                                       T1, src, dst)
    h = _tc_h(n, nhid, br)(Sp[0] + Sp[1], T1, dis1[:, None], be[None, :])

    hp = jax.lax.bitcast_convert_type(
        h.astype(jnp.bfloat16).reshape(n, nhid // 2, 2), jnp.int32)
    ew2, d2p = _sc_edges(n, ech, nhid)(zn, hp, src, dst)
    deg2 = d2p[0, 0] + d2p[1, 0]
    dis2 = jnp.where(deg2 > 0, lax.rsqrt(jnp.where(deg2 > 0, deg2, 1.0)), 0.0)

    T2 = XW2 * dis2[:, None]
    Up = _sc_spmm(n, ech, d2, True)(jnp.zeros((n, d2), jnp.float32),
                                    T2, src, dst, ew2)
    H1 = jax.nn.relu(dis2[:, None] * (Up[0] + Up[1]) + b_ab)
    Z = _tc_mm(n, d2, dz, br)(H1, W2blk)
    T3 = Z * dis2[:, None]
    Vp = _sc_spmm(n, ech, dz, True)(jnp.zeros((n, dz), jnp.float32),
                                    T3, src, dst, ew2)
    outk = dis2[:, None] * (Vp[0] + Vp[1])
    out1 = outk[:, :ncls] + b2a
    out2 = outk[:, ncls:2 * ncls] + b2b
    return (out1, out2)


# speedup vs baseline: 28.6949x; 1.0006x over previous
import functools

import jax
import jax.numpy as jnp
from jax import lax
from jax.experimental import pallas as pl
from jax.experimental.pallas import tpu as pltpu
from jax.experimental.pallas import tpu_sc as plsc

NC = 2
NS = 16
NW = NC * NS
L = 16
CH = 80
TAU = 0.05

_SC_PARAMS = pltpu.CompilerParams(use_tc_tiling_on_sc=False,
                                  needs_layout_passes=False)


@functools.cache
def _mesh():
    return plsc.VectorSubcoreMesh(core_axis_name="c", subcore_axis_name="s",
                                  num_cores=NC, num_subcores=NS)


def _f32(*shape):
    return jax.ShapeDtypeStruct(shape, jnp.float32)


@functools.cache
def _sc_deg(n, ech):
    cpw = ech // NW

    @functools.partial(
        pl.kernel,
        out_type=_f32(NC, 1, n),
        mesh=_mesh(),
        compiler_params=_SC_PARAMS,
        scratch_types=[
            pltpu.VMEM((cpw, CH), jnp.int32),
            pltpu.VMEM((CH,), jnp.float32),
            pltpu.VMEM_SHARED((n,), jnp.float32),
        ],
    )
    def deg_kernel(zeros_hbm, dst_hbm, degp_hbm, dstm, ones_v, acc):
        c = lax.axis_index("c")
        s = lax.axis_index("s")
        wid = s * NC + c
        for i in range(CH // L):
            ones_v[pl.ds(i * L, L)] = jnp.full((L,), 1.0, jnp.float32)

        @pl.when(s == 0)
        def _():
            pltpu.sync_copy(zeros_hbm, acc)

        plsc.subcore_barrier()
        pltpu.sync_copy(dst_hbm.at[wid], dstm)

        def chunk(j, carry):
            pltpu.sync_copy(ones_v, acc.at[dstm.at[j]], add=True)
            return carry

        lax.fori_loop(0, cpw, chunk, 0)
        plsc.subcore_barrier()

        @pl.when(s == 0)
        def _():
            pltpu.sync_copy(acc, degp_hbm.at[c, 0])

    return deg_kernel


@functools.cache
def _sc_spmm(n, ech, d, weighted):
    cpw = ech // NW
    rpn = n // NS

    assert cpw % 2 == 1
    npair = (cpw - 1) // 2
    scratch = [
        pltpu.VMEM((cpw, CH), jnp.int32),
        pltpu.VMEM((cpw, CH), jnp.int32),
        pltpu.VMEM((CH, d), jnp.float32),
        pltpu.VMEM((CH, d), jnp.float32),
        pltpu.VMEM_SHARED((n, d), jnp.float32),
        pltpu.SemaphoreType.DMA,
        pltpu.SemaphoreType.DMA,
        pltpu.SemaphoreType.DMA,
        pltpu.SemaphoreType.DMA,
    ]
    if weighted:
        scratch.insert(2, pltpu.VMEM((cpw, CH), jnp.float32))

    @functools.partial(pl.kernel, out_type=_f32(NC, n, d), mesh=_mesh(),
                       compiler_params=_SC_PARAMS, scratch_types=scratch)
    def spmm_kernel(*refs):
        if weighted:
            (zeros_hbm, table_hbm, src_hbm, dst_hbm, w_hbm, part_hbm,
             srcm, dstm, wm, rows0, rows1, acc,
             gsem0, gsem1, ssem0, ssem1) = refs
        else:
            (zeros_hbm, table_hbm, src_hbm, dst_hbm, part_hbm,
             srcm, dstm, rows0, rows1, acc,
             gsem0, gsem1, ssem0, ssem1) = refs
        c = lax.axis_index("c")
        s = lax.axis_index("s")
        wid = s * NC + c
        bufs = (rows0, rows1)
        gsems = (gsem0, gsem1)
        ssems = (ssem0, ssem1)

        @pl.when(s == 0)
        def _():
            pltpu.sync_copy(zeros_hbm, acc)

        plsc.subcore_barrier()
        pltpu.sync_copy(src_hbm.at[wid], srcm)
        pltpu.sync_copy(dst_hbm.at[wid], dstm)
        if weighted:
            pltpu.sync_copy(w_hbm.at[wid], wm)

        def start(j, b):
            pltpu.async_copy(table_hbm.at[srcm.at[j]], bufs[b], gsems[b])

        def finish(j, b):
            pltpu.make_async_copy(table_hbm.at[srcm.at[j]], bufs[b],
                                  gsems[b]).wait()

        def scale(j, b):
            rows = bufs[b]
            if weighted:
                def gscale(g, gcarry):
                    wvec = wm[j, pl.ds(g * L, L)]
                    for j2 in range(L):
                        e = g * L + j2
                        wv = wvec[j2]
                        for f in range(d // L):
                            rows[e, pl.ds(f * L, L)] = (
                                rows[e, pl.ds(f * L, L)] * wv)
                    return gcarry
                lax.fori_loop(0, CH // L, gscale, 0)

        def start_scatter(j, b):
            pltpu.async_copy(bufs[b], acc.at[dstm.at[j]], ssems[b], add=True)

        def wait_scatter(j, b):
            pltpu.make_async_copy(bufs[b], acc.at[dstm.at[j]],
                                  ssems[b]).wait()

        start(0, 0)
        start(1, 1)

        def pair(k, carry):
            j0 = 2 * k
            finish(j0, 0)
            scale(j0, 0)
            start_scatter(j0, 0)
            finish(j0 + 1, 1)
            scale(j0 + 1, 1)
            start_scatter(j0 + 1, 1)
            wait_scatter(j0, 0)
            start(j0 + 2, 0)
            wait_scatter(j0 + 1, 1)

            @pl.when(k < npair - 1)
            def _():
                start(j0 + 3, 1)

            return carry

        lax.fori_loop(0, npair, pair, 0)
        finish(cpw - 1, 0)
        scale(cpw - 1, 0)
        pltpu.sync_copy(bufs[0], acc.at[dstm.at[cpw - 1]], add=True)
        plsc.subcore_barrier()
        low = n - (NS - 1) * (rpn - 1)

        @pl.when(s < NS - 1)
        def _():
            pltpu.sync_copy(acc.at[pl.ds(s * (rpn - 1), rpn - 1)],
                            part_hbm.at[c, pl.ds(s * (rpn - 1), rpn - 1)])

        @pl.when(s == NS - 1)
        def _():
            pltpu.sync_copy(acc.at[pl.ds((NS - 1) * (rpn - 1), low)],
                            part_hbm.at[c, pl.ds((NS - 1) * (rpn - 1), low)])

    return spmm_kernel


@functools.cache
def _sc_edges(n, ech, d):
    cpw = ech // NW
    dp = d // 2

    @functools.partial(
        pl.kernel,
        out_type=(_f32(NW, cpw, CH), _f32(NC, 1, n)),
        mesh=_mesh(),
        compiler_params=_SC_PARAMS,
        scratch_types=[
            pltpu.VMEM((cpw, CH), jnp.int32),
            pltpu.VMEM((cpw, CH), jnp.int32),
            pltpu.VMEM((CH, dp), jnp.int32),
            pltpu.VMEM((CH, dp), jnp.int32),
            pltpu.VMEM((CH, dp), jnp.int32),
            pltpu.VMEM((CH, dp), jnp.int32),
            pltpu.VMEM((cpw, CH), jnp.float32),
            pltpu.VMEM((L * 17,), jnp.float32),
            pltpu.VMEM_SHARED((n,), jnp.float32),
            pltpu.SemaphoreType.DMA,
            pltpu.SemaphoreType.DMA,
        ],
    )
    def edges_kernel(zeros_hbm, h_hbm, src_hbm, dst_hbm, ew_hbm, degp_hbm,
                     srcm, dstm, hs0, hd0, hs1, hd1, ewm, tbuf, dacc,
                     sem0, sem1):
        c = lax.axis_index("c")
        s = lax.axis_index("s")
        wid = s * NC + c
        sbufs = (hs0, hs1)
        dbufs = (hd0, hd1)
        sems = (sem0, sem1)

        @pl.when(s == 0)
        def _():
            pltpu.sync_copy(zeros_hbm, dacc)

        plsc.subcore_barrier()
        pltpu.sync_copy(src_hbm.at[wid], srcm)
        pltpu.sync_copy(dst_hbm.at[wid], dstm)

        lane = jnp.arange(L, dtype=jnp.int32)

        def start(j, b):
            pltpu.async_copy(h_hbm.at[srcm.at[j]], sbufs[b], sems[b])
            pltpu.async_copy(h_hbm.at[dstm.at[j]], dbufs[b], sems[b])

        def finish(j, b):
            pltpu.make_async_copy(h_hbm.at[srcm.at[j]], sbufs[b],
                                  sems[b]).wait()
            pltpu.make_async_copy(h_hbm.at[dstm.at[j]], dbufs[b],
                                  sems[b]).wait()

        lane17 = lane * 17

        def process(j, b):
            hs = sbufs[b]
            hd = dbufs[b]

            def grp(g, gcarry):
                for j2 in range(L):
                    e = g * L + j2
                    p = None
                    for f in range(dp // L):
                        sa, sb = plsc.unpack(
                            plsc.bitcast(hs[e, pl.ds(f * L, L)],
                                         jnp.bfloat16),
                            format=plsc.PackFormat.INTERLEAVED)
                        da, db = plsc.unpack(
                            plsc.bitcast(hd[e, pl.ds(f * L, L)],
                                         jnp.bfloat16),
                            format=plsc.PackFormat.INTERLEAVED)
                        q = sa * da + sb * db
                        p = q if p is None else p + q
                    plsc.store_scatter(tbuf, [lane17 + j2], p)
                acc = plsc.load_gather(tbuf, [lane])
                for k in range(1, L):
                    acc = acc + plsc.load_gather(tbuf, [lane + k * 17])
                acc = jnp.where(acc >= TAU, acc, 0.0)
                ewm[j, pl.ds(g * L, L)] = acc
                return gcarry

            lax.fori_loop(0, CH // L, grp, 0)
            pltpu.sync_copy(ewm.at[j], dacc.at[dstm.at[j]], add=True)

        start(0, 0)

        def pair(k, carry):
            j0 = 2 * k
            start(j0 + 1, 1)
            finish(j0, 0)
            process(j0, 0)
            start(j0 + 2, 0)
            finish(j0 + 1, 1)
            process(j0 + 1, 1)
            return carry

        lax.fori_loop(0, (cpw - 1) // 2, pair, 0)
        finish(cpw - 1, 0)
        process(cpw - 1, 0)
        pltpu.sync_copy(ewm, ew_hbm.at[wid])
        plsc.subcore_barrier()

        @pl.when(s == 0)
        def _():
            pltpu.sync_copy(dacc, degp_hbm.at[c, 0])

    return edges_kernel


@functools.cache
def _tc_mm2(n, k, m1, m2, br):
    def body(x_ref, w1_ref, w2_ref, o1_ref, o2_ref):
        xb = x_ref[...]
        o1_ref[...] = jnp.dot(xb, w1_ref[...], preferred_element_type=jnp.float32)
        o2_ref[...] = jnp.dot(xb, w2_ref[...], preferred_element_type=jnp.float32)

    return pl.pallas_call(
        body,
        grid=(n // br,),
        in_specs=[
            pl.BlockSpec((br, k), lambda i: (i, 0)),
            pl.BlockSpec((k, m1), lambda i: (0, 0)),
            pl.BlockSpec((k, m2), lambda i: (0, 0)),
        ],
        out_specs=[
            pl.BlockSpec((br, m1), lambda i: (i, 0)),
            pl.BlockSpec((br, m2), lambda i: (i, 0)),
        ],
        out_shape=[_f32(n, m1), _f32(n, m2)],
    )


@functools.cache
def _tc_mm(n, k, m, br):
    def body(x_ref, w_ref, o_ref):
        o_ref[...] = jnp.dot(x_ref[...], w_ref[...],
                             preferred_element_type=jnp.float32)

    return pl.pallas_call(
        body,
        grid=(n // br,),
        in_specs=[
            pl.BlockSpec((br, k), lambda i: (i, 0)),
            pl.BlockSpec((k, m), lambda i: (0, 0)),
        ],
        out_specs=pl.BlockSpec((br, m), lambda i: (i, 0)),
        out_shape=_f32(n, m),
    )


@functools.cache
def _tc_h(n, d, br):
    def body(s_ref, t_ref, dis_ref, b_ref, o_ref):
        h = dis_ref[...] * (s_ref[...] + t_ref[...]) + b_ref[...]
        h = jnp.maximum(h, 0.0)
        hn = jnp.sqrt(jnp.sum(h * h, axis=1, keepdims=True))
        o_ref[...] = h / jnp.clip(hn, 1e-12, None)

    return pl.pallas_call(
        body,
        grid=(n // br,),
        in_specs=[
            pl.BlockSpec((br, d), lambda i: (i, 0)),
            pl.BlockSpec((br, d), lambda i: (i, 0)),
            pl.BlockSpec((br, 1), lambda i: (i, 0)),
            pl.BlockSpec((1, d), lambda i: (0, 0)),
        ],
        out_specs=pl.BlockSpec((br, d), lambda i: (i, 0)),
        out_shape=_f32(n, d),
    )


def kernel(x, edge_index, We, be, W1a, b1a, W2a, b2a, W1b, b1b, W2b, b2b):
    n, nfeat = x.shape
    e = edge_index.shape[1]
    nhid = We.shape[1]
    ncls = W2a.shape[1]
    ech = e // CH
    cpw = ech // NW
    br = 10000

    src = edge_index[0].astype(jnp.int32).reshape(NW, cpw, CH)
    dst = edge_index[1].astype(jnp.int32).reshape(NW, cpw, CH)

    W1ab = jnp.concatenate([W1a, W1b], axis=1)
    b_ab = jnp.concatenate([b1a, b1b])
    d2 = 2 * nhid
    dz = 32
    W2blk = jnp.zeros((d2, dz), jnp.float32)
    W2blk = W2blk.at[:nhid, :ncls].set(W2a).at[nhid:, ncls:2 * ncls].set(W2b)

    zn = jnp.zeros((n,), jnp.float32)

    XW, XW2 = _tc_mm2(n, nfeat, nhid, d2, br)(x, We, W1ab)

    degp = _sc_deg(n, ech)(zn, dst)
    dis1 = lax.rsqrt(degp[0, 0] + degp[1, 0] + 1.0)
    T1 = XW * dis1[:, None]
    Sp = _sc_spmm(n, ech, nhid, False)(jnp.zeros((n, nhid), jnp.float32),
                                       T1, src, dst)
    h = _tc_h(n, nhid, br)(Sp[0] + Sp[1], T1, dis1[:, None], be[None, :])

    hp = jax.lax.bitcast_convert_type(
        h.astype(jnp.bfloat16).reshape(n, nhid // 2, 2), jnp.int32)
    ew2, d2p = _sc_edges(n, ech, nhid)(zn, hp, src, dst)
    deg2 = d2p[0, 0] + d2p[1, 0]
    dis2 = jnp.where(deg2 > 0, lax.rsqrt(jnp.where(deg2 > 0, deg2, 1.0)), 0.0)

    T2 = XW2 * dis2[:, None]
    Up = _sc_spmm(n, ech, d2, True)(jnp.zeros((n, d2), jnp.float32),
                                    T2, src, dst, ew2)
    H1 = jax.nn.relu(dis2[:, None] * (Up[0] + Up[1]) + b_ab)
    Z = _tc_mm(n, d2, dz, br)(H1, W2blk)
    T3 = Z * dis2[:, None]
    Vp = _sc_spmm(n, ech, dz, True)(jnp.zeros((n, dz), jnp.float32),
                                    T3, src, dst, ew2)
    outk = dis2[:, None] * (Vp[0] + Vp[1])
    out1 = outk[:, :ncls] + b2a
    out2 = outk[:, ncls:2 * ncls] + b2b
    return (out1, out2)
